# baseline (speedup 1.0000x reference)
import jax
import jax.numpy as jnp
from jax import lax
from jax.experimental import pallas as pl
from jax.experimental.pallas import tpu as pltpu
from jax.experimental.pallas import tpu_sc as plsc

NS = 16
CH = 128
NROW = 2
GLA = 1
NIDX = 4
ZR = 128


def _relu_matmul_table(x, rel_weights):
    n, d_in = x.shape
    r, _, d_out = rel_weights.shape
    bn = 2000
    assert n % bn == 0

    def body(x_ref, w_ref, o_ref):
        o_ref[...] = jnp.maximum(
            jnp.dot(x_ref[...], w_ref[0], preferred_element_type=jnp.float32),
            0.0,
        )[None]

    return pl.pallas_call(
        body,
        grid=(r, n // bn),
        in_specs=[
            pl.BlockSpec((bn, d_in), lambda ri, i: (i, 0)),
            pl.BlockSpec((1, d_in, d_out), lambda ri, i: (ri, 0, 0)),
        ],
        out_specs=pl.BlockSpec((1, bn, d_out), lambda ri, i: (ri, i, 0)),
        out_shape=jax.ShapeDtypeStruct((r, n, d_out), jnp.float32),
    )(x, rel_weights)


def _sc_gather_scatter(table, eidx, zinit, n_nodes, nch):
    d = table.shape[1]
    rpa = -(-(-(-n_nodes // NS)) // ZR) * ZR
    n_acc = NS * rpa
    last = n_nodes - (NS - 1) * rpa
    assert n_acc > n_nodes and 0 < last <= rpa and last % 8 == 0
    assert nch % NIDX == 0 and nch >= NIDX
    mesh = plsc.VectorSubcoreMesh(
        core_axis_name="c", subcore_axis_name="s",
        num_cores=2, num_subcores=NS)

    def body(table_hbm, eidx_hbm, zinit_hbm, out_hbm,
             idx_v, rows_v, acc, sem_i, sem_g, sem_s):
        cid = lax.axis_index("c")
        sid = lax.axis_index("s")

        @pl.when(cid == 0)
        def _core0_body():
            _tile_work(table_hbm, eidx_hbm, zinit_hbm, out_hbm,
                       idx_v, rows_v, acc, sem_i, sem_g, sem_s, sid)

    def _tile_work(table_hbm, eidx_hbm, zinit_hbm, out_hbm,
                   idx_v, rows_v, acc, sem_i, sem_g, sem_s, sid):
        r0 = sid * rpa
        for k in range(rpa // ZR):
            pltpu.sync_copy(zinit_hbm, acc.at[pl.ds(r0 + k * ZR, ZR)])

        plsc.subcore_barrier()

        def fetch_idx(c, b):
            pltpu.async_copy(eidx_hbm.at[sid, c], idx_v.at[b], sem_i.at[b])

        def fetch_idx_wait(c, b):
            pltpu.make_async_copy(eidx_hbm.at[sid, c], idx_v.at[b],
                                  sem_i.at[b]).wait()

        def gather(b, rb):
            pltpu.async_copy(table_hbm.at[idx_v.at[b, 0]], rows_v.at[rb],
                             sem_g.at[rb])

        def gather_wait(b, rb):
            pltpu.make_async_copy(table_hbm.at[idx_v.at[b, 0]],
                                  rows_v.at[rb], sem_g.at[rb]).wait()

        def scatter(b, rb):
            pltpu.async_copy(rows_v.at[rb], acc.at[idx_v.at[b, 1]],
                             sem_s.at[rb], add=True)

        def scatter_wait(b, rb):
            pltpu.make_async_copy(rows_v.at[rb], acc.at[idx_v.at[b, 1]],
                                  sem_s.at[rb]).wait()

        for j in range(GLA + 2):
            fetch_idx(j, j % NIDX)
        for j in range(GLA):
            fetch_idx_wait(j, j % NIDX)
            gather(j % NIDX, j % NROW)

        @pl.loop(0, nch, step=NIDX)
        def _(c0):
            for k in range(NIDX):
                cc = c0 + k
                b = k
                rb = k % NROW
                gb = (k + GLA) % NIDX
                grb = (k + GLA) % NROW
                fb = (k + GLA + 2) % NIDX

                @pl.when((cc >= 1) & (cc + GLA < nch))
                def _():
                    scatter_wait((k - 1) % NIDX, (k - 1) % NROW)

                @pl.when(cc + GLA < nch)
                def _():
                    fetch_idx_wait(cc + GLA, gb)
                    gather(gb, grb)

                @pl.when(cc + GLA + 2 < nch)
                def _():
                    fetch_idx(cc + GLA + 2, fb)

                gather_wait(b, rb)
                scatter(b, rb)

        for j in range(nch - NROW, nch):
            scatter_wait(j % NIDX, j % NROW)

        plsc.subcore_barrier()

        @pl.when(sid < NS - 1)
        def _():
            pltpu.sync_copy(acc.at[pl.ds(r0, rpa)],
                            out_hbm.at[pl.ds(r0, rpa)])

        @pl.when(sid == NS - 1)
        def _():
            pltpu.sync_copy(acc.at[pl.ds(r0, last)],
                            out_hbm.at[pl.ds(r0, last)])

    return pl.kernel(
        body,
        out_type=jax.ShapeDtypeStruct((n_nodes, d), jnp.float32),
        mesh=mesh,
        scratch_types=[
            pltpu.VMEM((NIDX, 2, CH), jnp.int32),
            pltpu.VMEM((NROW, CH, d), jnp.float32),
            pltpu.VMEM_SHARED((n_acc, d), jnp.float32),
            pltpu.SemaphoreType.DMA((NIDX,)),
            pltpu.SemaphoreType.DMA((NROW,)),
            pltpu.SemaphoreType.DMA((NROW,)),
        ],
    )(table, eidx, zinit)


def kernel(x, edge_index, edge_type, rel_weights):
    n_nodes, d_in = x.shape
    n_rel, _, d_out = rel_weights.shape
    n_edges = edge_index.shape[1]

    dst = edge_index[0].astype(jnp.int32)
    src = edge_index[1].astype(jnp.int32)
    et = edge_type.astype(jnp.int32)
    gidx = et * n_nodes + src

    t_chunks = -(-n_edges // CH)
    nch = -(-t_chunks // NS)
    nch += (-nch) % NIDX
    e_pad = NS * nch * CH
    pad = e_pad - n_edges
    assert pad >= 0
    if pad:
        gidx = jnp.concatenate([gidx, jnp.zeros((pad,), jnp.int32)])
        dst = jnp.concatenate([dst, jnp.full((pad,), n_nodes, jnp.int32)])
    eidx = jnp.stack(
        [gidx.reshape(NS, nch, CH), dst.reshape(NS, nch, CH)], axis=2)

    table = _relu_matmul_table(x, rel_weights).reshape(n_rel * n_nodes, d_out)
    zinit = jnp.zeros((ZR, d_out), jnp.float32)
    return _sc_gather_scatter(table, eidx, zinit, n_nodes, nch)


# speedup vs baseline: 1.4770x; 1.4770x over previous
import jax
import jax.numpy as jnp
from jax import lax
from jax.experimental import pallas as pl
from jax.experimental.pallas import tpu as pltpu
from jax.experimental.pallas import tpu_sc as plsc

NS = 16
CH = 120
NROW = 3
SLAG = 2
NIDX = 6
ZR = 128


def _relu_matmul_table(x, rel_weights):
    n, d_in = x.shape
    r, _, d_out = rel_weights.shape
    bn = 2000
    assert n % bn == 0

    def body(x_ref, w_ref, o_ref):
        o_ref[...] = jnp.maximum(
            jnp.dot(x_ref[...], w_ref[0], preferred_element_type=jnp.float32),
            0.0,
        )[None]

    return pl.pallas_call(
        body,
        grid=(r, n // bn),
        in_specs=[
            pl.BlockSpec((bn, d_in), lambda ri, i: (i, 0)),
            pl.BlockSpec((1, d_in, d_out), lambda ri, i: (ri, 0, 0)),
        ],
        out_specs=pl.BlockSpec((1, bn, d_out), lambda ri, i: (ri, i, 0)),
        out_shape=jax.ShapeDtypeStruct((r, n, d_out), jnp.float32),
    )(x, rel_weights)


def _sc_gather_scatter(table, eidx, zinit, n_nodes, nch):
    d = table.shape[1]
    rpa = -(-(-(-n_nodes // NS)) // ZR) * ZR
    n_acc = NS * rpa
    last = n_nodes - (NS - 1) * rpa
    assert n_acc > n_nodes and 0 < last <= rpa and last % 8 == 0
    assert nch % NIDX == 0 and nch >= NIDX
    mesh = plsc.VectorSubcoreMesh(
        core_axis_name="c", subcore_axis_name="s",
        num_cores=2, num_subcores=NS)

    def body(table_hbm, eidx_hbm, zinit_hbm, out_hbm,
             idx_v, rows_v, acc, sem_i, sem_g, sem_s):
        cid = lax.axis_index("c")
        sid = lax.axis_index("s")

        @pl.when(cid == 0)
        def _core0_body():
            _tile_work(table_hbm, eidx_hbm, zinit_hbm, out_hbm,
                       idx_v, rows_v, acc, sem_i, sem_g, sem_s, sid)

    def _tile_work(table_hbm, eidx_hbm, zinit_hbm, out_hbm,
                   idx_v, rows_v, acc, sem_i, sem_g, sem_s, sid):
        r0 = sid * rpa
        for k in range(rpa // ZR):
            pltpu.sync_copy(zinit_hbm, acc.at[pl.ds(r0 + k * ZR, ZR)])

        plsc.subcore_barrier()

        def fetch_idx(c, b):
            pltpu.async_copy(eidx_hbm.at[sid, c], idx_v.at[b], sem_i.at[b])

        def fetch_idx_wait(c, b):
            pltpu.make_async_copy(eidx_hbm.at[sid, c], idx_v.at[b],
                                  sem_i.at[b]).wait()

        def gather(b, rb):
            pltpu.async_copy(table_hbm.at[idx_v.at[b, 0]], rows_v.at[rb],
                             sem_g.at[rb])

        def gather_wait(b, rb):
            pltpu.make_async_copy(table_hbm.at[idx_v.at[b, 0]],
                                  rows_v.at[rb], sem_g.at[rb]).wait()

        def scatter(b, rb):
            pltpu.async_copy(rows_v.at[rb], acc.at[idx_v.at[b, 1]],
                             sem_s.at[rb], add=True)

        def scatter_wait(b, rb):
            pltpu.make_async_copy(rows_v.at[rb], acc.at[idx_v.at[b, 1]],
                                  sem_s.at[rb]).wait()

        for j in range(3):
            fetch_idx(j, j % NIDX)
        fetch_idx_wait(0, 0)
        gather(0, 0)

        @pl.loop(0, nch, step=NIDX)
        def _(c0):
            for k in range(NIDX):
                cc = c0 + k
                b = k
                rb = k % NROW
                gb = (k + 1) % NIDX
                grb = (k + 1) % NROW
                fb = (k + 3) % NIDX

                @pl.when(cc + 1 < nch)
                def _():
                    fetch_idx_wait(cc + 1, gb)

                    @pl.when(cc >= SLAG)
                    def _():
                        scatter_wait((k - SLAG) % NIDX, (k - SLAG) % NROW)

                    gather(gb, grb)

                @pl.when(cc + 3 < nch)
                def _():
                    fetch_idx(cc + 3, fb)

                gather_wait(b, rb)
                scatter(b, rb)

        for j in range(nch - NROW, nch):
            scatter_wait(j % NIDX, j % NROW)

        plsc.subcore_barrier()

        @pl.when(sid < NS - 1)
        def _():
            pltpu.sync_copy(acc.at[pl.ds(r0, rpa)],
                            out_hbm.at[pl.ds(r0, rpa)])

        @pl.when(sid == NS - 1)
        def _():
            pltpu.sync_copy(acc.at[pl.ds(r0, last)],
                            out_hbm.at[pl.ds(r0, last)])

    return pl.kernel(
        body,
        out_type=jax.ShapeDtypeStruct((n_nodes, d), jnp.float32),
        mesh=mesh,
        scratch_types=[
            pltpu.VMEM((NIDX, 2, CH), jnp.int32),
            pltpu.VMEM((NROW, CH, d), jnp.float32),
            pltpu.VMEM_SHARED((n_acc, d), jnp.float32),
            pltpu.SemaphoreType.DMA((NIDX,)),
            pltpu.SemaphoreType.DMA((NROW,)),
            pltpu.SemaphoreType.DMA((NROW,)),
        ],
    )(table, eidx, zinit)


def kernel(x, edge_index, edge_type, rel_weights):
    n_nodes, d_in = x.shape
    n_rel, _, d_out = rel_weights.shape
    n_edges = edge_index.shape[1]

    dst = edge_index[0].astype(jnp.int32)
    src = edge_index[1].astype(jnp.int32)
    et = edge_type.astype(jnp.int32)
    gidx = et * n_nodes + src

    t_chunks = -(-n_edges // CH)
    nch = -(-t_chunks // NS)
    nch += (-nch) % NIDX
    e_pad = NS * nch * CH
    pad = e_pad - n_edges
    assert pad >= 0
    if pad:
        gidx = jnp.concatenate([gidx, jnp.zeros((pad,), jnp.int32)])
        dst = jnp.concatenate([dst, jnp.full((pad,), n_nodes, jnp.int32)])
    eidx = jnp.stack(
        [gidx.reshape(NS, nch, CH), dst.reshape(NS, nch, CH)], axis=2)

    table = _relu_matmul_table(x, rel_weights).reshape(n_rel * n_nodes, d_out)
    zinit = jnp.zeros((ZR, d_out), jnp.float32)
    return _sc_gather_scatter(table, eidx, zinit, n_nodes, nch)


# speedup vs baseline: 1.5256x; 1.0329x over previous
import jax
import jax.numpy as jnp
from jax import lax
from jax.experimental import pallas as pl
from jax.experimental.pallas import tpu as pltpu
from jax.experimental.pallas import tpu_sc as plsc

NS = 16
CH = 120
NROW = 3
SLAG = 2
NIDX = 6
ZR = 128


def _relu_matmul_table(x, rel_weights):
    n, d_in = x.shape
    r, _, d_out = rel_weights.shape
    bn = 2000
    assert n % bn == 0

    def body(x_ref, w_ref, o_ref):
        o_ref[...] = jnp.maximum(
            jnp.dot(x_ref[...], w_ref[0], preferred_element_type=jnp.float32),
            0.0,
        )[None]

    return pl.pallas_call(
        body,
        grid=(n // bn, r),
        in_specs=[
            pl.BlockSpec((bn, d_in), lambda i, ri: (i, 0)),
            pl.BlockSpec((1, d_in, d_out), lambda i, ri: (ri, 0, 0)),
        ],
        out_specs=pl.BlockSpec((1, bn, d_out), lambda i, ri: (ri, i, 0)),
        out_shape=jax.ShapeDtypeStruct((r, n, d_out), jnp.float32),
    )(x, rel_weights)


def _sc_gather_scatter(table, eidx, zinit, n_nodes, nch):
    d = table.shape[1]
    rpa = -(-(-(-n_nodes // NS)) // ZR) * ZR
    n_acc = NS * rpa
    last = n_nodes - (NS - 1) * rpa
    assert n_acc > n_nodes and 0 < last <= rpa and last % 8 == 0
    assert nch % NIDX == 0 and nch >= NIDX
    mesh = plsc.VectorSubcoreMesh(
        core_axis_name="c", subcore_axis_name="s",
        num_cores=2, num_subcores=NS)

    def body(table_hbm, eidx_hbm, zinit_hbm, out_hbm,
             idx_v, rows_v, acc, sem_i, sem_g, sem_s):
        cid = lax.axis_index("c")
        sid = lax.axis_index("s")

        @pl.when(cid == 0)
        def _core0_body():
            _tile_work(table_hbm, eidx_hbm, zinit_hbm, out_hbm,
                       idx_v, rows_v, acc, sem_i, sem_g, sem_s, sid)

    def _tile_work(table_hbm, eidx_hbm, zinit_hbm, out_hbm,
                   idx_v, rows_v, acc, sem_i, sem_g, sem_s, sid):
        r0 = sid * rpa
        for k in range(rpa // ZR):
            pltpu.sync_copy(zinit_hbm, acc.at[pl.ds(r0 + k * ZR, ZR)])

        plsc.subcore_barrier()

        def fetch_idx(c, b):
            pltpu.async_copy(eidx_hbm.at[sid, c], idx_v.at[b], sem_i.at[b])

        def fetch_idx_wait(c, b):
            pltpu.make_async_copy(eidx_hbm.at[sid, c], idx_v.at[b],
                                  sem_i.at[b]).wait()

        def gather(b, rb):
            pltpu.async_copy(table_hbm.at[idx_v.at[b, 0]], rows_v.at[rb],
                             sem_g.at[rb])

        def gather_wait(b, rb):
            pltpu.make_async_copy(table_hbm.at[idx_v.at[b, 0]],
                                  rows_v.at[rb], sem_g.at[rb]).wait()

        def scatter(b, rb):
            pltpu.async_copy(rows_v.at[rb], acc.at[idx_v.at[b, 1]],
                             sem_s.at[rb], add=True)

        def scatter_wait(b, rb):
            pltpu.make_async_copy(rows_v.at[rb], acc.at[idx_v.at[b, 1]],
                                  sem_s.at[rb]).wait()

        for j in range(3):
            fetch_idx(j, j % NIDX)
        fetch_idx_wait(0, 0)
        gather(0, 0)

        @pl.loop(0, nch, step=NIDX)
        def _(c0):
            for k in range(NIDX):
                cc = c0 + k
                b = k
                rb = k % NROW
                gb = (k + 1) % NIDX
                grb = (k + 1) % NROW
                fb = (k + 3) % NIDX

                @pl.when(cc + 1 < nch)
                def _():
                    fetch_idx_wait(cc + 1, gb)

                    @pl.when(cc >= SLAG)
                    def _():
                        scatter_wait((k - SLAG) % NIDX, (k - SLAG) % NROW)

                    gather(gb, grb)

                @pl.when(cc + 3 < nch)
                def _():
                    fetch_idx(cc + 3, fb)

                gather_wait(b, rb)
                scatter(b, rb)

        for j in range(nch - NROW, nch):
            scatter_wait(j % NIDX, j % NROW)

        plsc.subcore_barrier()

        @pl.when(sid < NS - 1)
        def _():
            pltpu.sync_copy(acc.at[pl.ds(r0, rpa)],
                            out_hbm.at[pl.ds(r0, rpa)])

        @pl.when(sid == NS - 1)
        def _():
            pltpu.sync_copy(acc.at[pl.ds(r0, last)],
                            out_hbm.at[pl.ds(r0, last)])

    return pl.kernel(
        body,
        out_type=jax.ShapeDtypeStruct((n_nodes, d), jnp.float32),
        mesh=mesh,
        scratch_types=[
            pltpu.VMEM((NIDX, 2, CH), jnp.int32),
            pltpu.VMEM((NROW, CH, d), jnp.float32),
            pltpu.VMEM_SHARED((n_acc, d), jnp.float32),
            pltpu.SemaphoreType.DMA((NIDX,)),
            pltpu.SemaphoreType.DMA((NROW,)),
            pltpu.SemaphoreType.DMA((NROW,)),
        ],
    )(table, eidx, zinit)


def kernel(x, edge_index, edge_type, rel_weights):
    n_nodes, d_in = x.shape
    n_rel, _, d_out = rel_weights.shape
    n_edges = edge_index.shape[1]

    dst = edge_index[0].astype(jnp.int32)
    src = edge_index[1].astype(jnp.int32)
    et = edge_type.astype(jnp.int32)
    gidx = et * n_nodes + src

    t_chunks = -(-n_edges // CH)
    nch = -(-t_chunks // NS)
    nch += (-nch) % NIDX
    e_pad = NS * nch * CH
    pad = e_pad - n_edges
    assert pad >= 0
    if pad:
        gidx = jnp.concatenate([gidx, jnp.zeros((pad,), jnp.int32)])
        dst = jnp.concatenate([dst, jnp.full((pad,), n_nodes, jnp.int32)])
    eidx = jnp.stack(
        [gidx.reshape(NS, nch, CH), dst.reshape(NS, nch, CH)], axis=2)

    table = _relu_matmul_table(x, rel_weights).reshape(n_rel * n_nodes, d_out)
    zinit = jnp.zeros((ZR, d_out), jnp.float32)
    return _sc_gather_scatter(table, eidx, zinit, n_nodes, nch)
